# Initial kernel scaffold; baseline (speedup 1.0000x reference)
import jax
import jax.numpy as jnp
from jax.experimental import pallas as pl


def kernel(x, edge_index, edge_attr):
    raise NotImplementedError("write your pallas kernel here")


# speedup vs baseline: 2.1747x; 2.1747x over previous
import functools

import jax
import jax.numpy as jnp
from jax import lax
from jax.experimental import pallas as pl
from jax.experimental.pallas import tpu as pltpu
from jax.experimental.pallas import tpu_sc as plsc

_GAMMA = 12.0
_D = 128
_CHUNK = 80

_info = plsc.get_sparse_core_info()
_NC = _info.num_cores
_NS = _info.num_subcores
_NW = _NC * _NS
_L = _info.num_lanes


def _normalize_body(x_ref, o_ref):
    x = x_ref[...]
    n = jnp.sqrt(jnp.sum(x * x, axis=1, keepdims=True))
    o_ref[...] = x / jnp.maximum(n, 1e-12)


def _normalize(x):
    n_rows, d = x.shape
    blk = 1000
    return pl.pallas_call(
        _normalize_body,
        grid=(n_rows // blk,),
        in_specs=[pl.BlockSpec((blk, d), lambda i: (i, 0))],
        out_specs=pl.BlockSpec((blk, d), lambda i: (i, 0)),
        out_shape=jax.ShapeDtypeStruct((n_rows, d), x.dtype),
    )(x)


def _finish_body(p_ref, o_ref):
    o_ref[...] = _GAMMA - jnp.sum(p_ref[...], axis=1, keepdims=True)


def _finish(partial):
    n_rows = partial.shape[0]
    blk = 2000
    return pl.pallas_call(
        _finish_body,
        grid=(n_rows // blk,),
        in_specs=[pl.BlockSpec((blk, _L), lambda i: (i, 0))],
        out_specs=pl.BlockSpec((blk, 1), lambda i: (i, 0)),
        out_shape=jax.ShapeDtypeStruct((n_rows, 1), jnp.float32),
    )(partial)


def _make_sc_partial(num_edges):
    per_w = num_edges // _NW
    n_chunks = per_w // _CHUNK
    assert per_w * _NW == num_edges and n_chunks * _CHUNK == per_w

    mesh = plsc.VectorSubcoreMesh(core_axis_name="c", subcore_axis_name="s")

    @functools.partial(
        pl.kernel,
        mesh=mesh,
        out_type=jax.ShapeDtypeStruct((num_edges, _L), jnp.float32),
        scratch_types=[
            pltpu.VMEM((_CHUNK,), jnp.int32),
            pltpu.VMEM((_CHUNK,), jnp.int32),
            pltpu.VMEM((_CHUNK, _D), jnp.float32),
            pltpu.VMEM((_CHUNK, _D), jnp.float32),
            pltpu.VMEM((_CHUNK, 2 * _D), jnp.float32),
            pltpu.VMEM((_CHUNK, _L), jnp.float32),
            pltpu.SemaphoreType.DMA,
            pltpu.SemaphoreType.DMA,
        ],
    )
    def sc_partial(xn, src, dst, attr, out, src_v, dst_v, head_v, tail_v,
                   rel_v, out_v, sem1, sem2):
        wid = lax.axis_index("s") * _NC + lax.axis_index("c")

        def chunk_body(j, carry):
            base = wid * per_w + j * _CHUNK
            pltpu.sync_copy(src.at[pl.ds(base, _CHUNK)], src_v)
            pltpu.sync_copy(dst.at[pl.ds(base, _CHUNK)], dst_v)
            cp1 = pltpu.async_copy(xn.at[src_v], head_v, sem1)
            cp2 = pltpu.async_copy(xn.at[dst_v], tail_v, sem2)
            pltpu.sync_copy(attr.at[pl.ds(base, _CHUNK)], rel_v)
            cp1.wait()
            cp2.wait()

            def edge_body(e, carry2):
                acc = jnp.zeros((_L,), jnp.float32)
                for jj in range(_D // _L):
                    h = head_v[e, pl.ds(jj * _L, _L)]
                    t = tail_v[e, pl.ds(jj * _L, _L)]
                    rh = rel_v[e, pl.ds(jj * _L, _L)]
                    rt = rel_v[e, pl.ds(_D + jj * _L, _L)]
                    acc = acc + jnp.abs(h * rh - t * rt)
                out_v[e, pl.ds(0, _L)] = acc
                return carry2

            lax.fori_loop(0, _CHUNK, edge_body, 0)
            pltpu.sync_copy(out_v, out.at[pl.ds(base, _CHUNK)])
            return carry

        lax.fori_loop(0, n_chunks, chunk_body, 0)

    return sc_partial


def kernel(x, edge_index, edge_attr):
    xn = _normalize(x.astype(jnp.float32))
    src = edge_index[0].astype(jnp.int32)
    dst = edge_index[1].astype(jnp.int32)
    partial = _make_sc_partial(src.shape[0])(xn, src, dst,
                                             edge_attr.astype(jnp.float32))
    return _finish(partial)


# speedup vs baseline: 2.8872x; 1.3276x over previous
import functools

import jax
import jax.numpy as jnp
from jax import lax
from jax.experimental import pallas as pl
from jax.experimental.pallas import tpu as pltpu
from jax.experimental.pallas import tpu_sc as plsc

_GAMMA = 12.0
_D = 128
_CHUNK = 80

_info = plsc.get_sparse_core_info()
_NC = _info.num_cores
_NS = _info.num_subcores
_NW = _NC * _NS
_L = _info.num_lanes


def _normalize_body(x_ref, o_ref):
    x = x_ref[...]
    n = jnp.sqrt(jnp.sum(x * x, axis=1, keepdims=True))
    o_ref[...] = x / jnp.maximum(n, 1e-12)


def _normalize(x):
    n_rows, d = x.shape
    blk = 1000
    return pl.pallas_call(
        _normalize_body,
        grid=(n_rows // blk,),
        in_specs=[pl.BlockSpec((blk, d), lambda i: (i, 0))],
        out_specs=pl.BlockSpec((blk, d), lambda i: (i, 0)),
        out_shape=jax.ShapeDtypeStruct((n_rows, d), x.dtype),
    )(x)


def _finish_body(p_ref, o_ref):
    o_ref[...] = _GAMMA - jnp.sum(p_ref[...], axis=1, keepdims=True)


def _finish(partial):
    n_rows = partial.shape[0]
    blk = 2000
    return pl.pallas_call(
        _finish_body,
        grid=(n_rows // blk,),
        in_specs=[pl.BlockSpec((blk, _L), lambda i: (i, 0))],
        out_specs=pl.BlockSpec((blk, 1), lambda i: (i, 0)),
        out_shape=jax.ShapeDtypeStruct((n_rows, 1), jnp.float32),
    )(partial)


def _make_sc_partial(num_edges):
    per_w = num_edges // _NW
    n_chunks = per_w // _CHUNK
    assert per_w * _NW == num_edges and n_chunks * _CHUNK == per_w
    assert n_chunks % 2 == 1

    mesh = plsc.VectorSubcoreMesh(core_axis_name="c", subcore_axis_name="s")

    buf_types = [
        pltpu.VMEM((_CHUNK,), jnp.int32),
        pltpu.VMEM((_CHUNK,), jnp.int32),
        pltpu.VMEM((_CHUNK, _D), jnp.float32),
        pltpu.VMEM((_CHUNK, _D), jnp.float32),
        pltpu.VMEM((_CHUNK, 2 * _D), jnp.float32),
        pltpu.VMEM((_CHUNK, _L), jnp.float32),
    ]
    sem_types = [pltpu.SemaphoreType.DMA] * 6

    @functools.partial(
        pl.kernel,
        mesh=mesh,
        out_type=jax.ShapeDtypeStruct((num_edges, _L), jnp.float32),
        scratch_types=buf_types + buf_types + sem_types,
    )
    def sc_partial(xn, src, dst, attr, out,
                   sv0, dv0, hv0, tv0, rv0, ov0,
                   sv1, dv1, hv1, tv1, rv1, ov1,
                   sh0, st0, sr0, sh1, st1, sr1):
        wid = lax.axis_index("s") * _NC + lax.axis_index("c")
        bufs = ((sv0, dv0, hv0, tv0, rv0, ov0, sh0, st0, sr0),
                (sv1, dv1, hv1, tv1, rv1, ov1, sh1, st1, sr1))

        def issue(jc, b):
            sv, dv, hv, tv, rv, _, sh, st, sr = bufs[b]
            base = wid * per_w + jc * _CHUNK
            pltpu.sync_copy(src.at[pl.ds(base, _CHUNK)], sv)
            pltpu.sync_copy(dst.at[pl.ds(base, _CHUNK)], dv)
            pltpu.async_copy(xn.at[sv], hv, sh)
            pltpu.async_copy(xn.at[dv], tv, st)
            pltpu.async_copy(attr.at[pl.ds(base, _CHUNK)], rv, sr)

        def wait_bufs(b):
            sv, dv, hv, tv, rv, _, sh, st, sr = bufs[b]
            pltpu.make_async_copy(xn.at[sv], hv, sh).wait()
            pltpu.make_async_copy(xn.at[dv], tv, st).wait()
            pltpu.make_async_copy(attr.at[pl.ds(0, _CHUNK)], rv, sr).wait()

        def compute(jc, b):
            _, _, hv, tv, rv, ov, _, _, _ = bufs[b]

            def edge_body(i2, carry2):
                for u in range(2):
                    e = i2 * 2 + u
                    acc = jnp.zeros((_L,), jnp.float32)
                    for jj in range(_D // _L):
                        h = hv[e, pl.ds(jj * _L, _L)]
                        t = tv[e, pl.ds(jj * _L, _L)]
                        rh = rv[e, pl.ds(jj * _L, _L)]
                        rt = rv[e, pl.ds(_D + jj * _L, _L)]
                        acc = acc + jnp.abs(h * rh - t * rt)
                    ov[e, pl.ds(0, _L)] = acc
                return carry2

            lax.fori_loop(0, _CHUNK // 2, edge_body, 0)
            base = wid * per_w + jc * _CHUNK
            pltpu.sync_copy(ov, out.at[pl.ds(base, _CHUNK)])

        issue(0, 0)

        def body(k, carry):
            jc = 2 * k
            wait_bufs(0)
            issue(jc + 1, 1)
            compute(jc, 0)
            wait_bufs(1)
            issue(jc + 2, 0)
            compute(jc + 1, 1)
            return carry

        lax.fori_loop(0, (n_chunks - 1) // 2, body, 0)
        wait_bufs(0)
        compute(n_chunks - 1, 0)

    return sc_partial


def kernel(x, edge_index, edge_attr):
    xn = _normalize(x.astype(jnp.float32))
    src = edge_index[0].astype(jnp.int32)
    dst = edge_index[1].astype(jnp.int32)
    partial = _make_sc_partial(src.shape[0])(xn, src, dst,
                                             edge_attr.astype(jnp.float32))
    return _finish(partial)


# speedup vs baseline: 3.2003x; 1.1084x over previous
import functools

import jax
import jax.numpy as jnp
from jax import lax
from jax.experimental import pallas as pl
from jax.experimental.pallas import tpu as pltpu
from jax.experimental.pallas import tpu_sc as plsc

_GAMMA = 12.0
_D = 128
_CHUNK = 80

_info = plsc.get_sparse_core_info()
_NC = _info.num_cores
_NS = _info.num_subcores
_NW = _NC * _NS
_L = _info.num_lanes


def _normalize_body(x_ref, o_ref):
    x = x_ref[...]
    n = jnp.sqrt(jnp.sum(x * x, axis=1, keepdims=True))
    o_ref[...] = x / jnp.maximum(n, 1e-12)


def _normalize(x):
    n_rows, d = x.shape
    blk = 1000
    return pl.pallas_call(
        _normalize_body,
        grid=(n_rows // blk,),
        in_specs=[pl.BlockSpec((blk, d), lambda i: (i, 0))],
        out_specs=pl.BlockSpec((blk, d), lambda i: (i, 0)),
        out_shape=jax.ShapeDtypeStruct((n_rows, d), x.dtype),
    )(x)


def _finish_body(p_ref, o_ref):
    o_ref[...] = _GAMMA - jnp.sum(p_ref[...], axis=1, keepdims=True)


def _finish(partial):
    n_rows = partial.shape[0]
    blk = 8000
    return pl.pallas_call(
        _finish_body,
        grid=(n_rows // blk,),
        in_specs=[pl.BlockSpec((blk, _L), lambda i: (i, 0))],
        out_specs=pl.BlockSpec((blk, 1), lambda i: (i, 0)),
        out_shape=jax.ShapeDtypeStruct((n_rows, 1), jnp.float32),
    )(partial)


def _make_sc_partial(num_edges):
    per_w = num_edges // _NW
    n_chunks = per_w // _CHUNK
    assert per_w * _NW == num_edges and n_chunks * _CHUNK == per_w
    assert n_chunks % 2 == 1

    mesh = plsc.VectorSubcoreMesh(core_axis_name="c", subcore_axis_name="s")

    buf_types = [
        pltpu.VMEM((_CHUNK,), jnp.int32),
        pltpu.VMEM((_CHUNK,), jnp.int32),
        pltpu.VMEM((_CHUNK, _D), jnp.float32),
        pltpu.VMEM((_CHUNK, _D), jnp.float32),
        pltpu.VMEM((_CHUNK, 2 * _D), jnp.float32),
        pltpu.VMEM((_CHUNK, _L), jnp.float32),
    ]
    sem_types = [pltpu.SemaphoreType.DMA] * 6

    @functools.partial(
        pl.kernel,
        mesh=mesh,
        out_type=jax.ShapeDtypeStruct((num_edges, _L), jnp.float32),
        scratch_types=buf_types + buf_types + sem_types,
    )
    def sc_partial(xn, ei, attr, out,
                   sv0, dv0, hv0, tv0, rv0, ov0,
                   sv1, dv1, hv1, tv1, rv1, ov1,
                   sh0, st0, sr0, sh1, st1, sr1):
        wid = lax.axis_index("s") * _NC + lax.axis_index("c")
        bufs = ((sv0, dv0, hv0, tv0, rv0, ov0, sh0, st0, sr0),
                (sv1, dv1, hv1, tv1, rv1, ov1, sh1, st1, sr1))

        def issue(jc, b):
            sv, dv, hv, tv, rv, _, sh, st, sr = bufs[b]
            base = wid * per_w + jc * _CHUNK
            pltpu.sync_copy(ei.at[pl.ds(base, _CHUNK)], sv)
            pltpu.sync_copy(ei.at[pl.ds(num_edges + base, _CHUNK)], dv)
            pltpu.async_copy(xn.at[sv], hv, sh)
            pltpu.async_copy(xn.at[dv], tv, st)
            pltpu.async_copy(attr.at[pl.ds(base, _CHUNK)], rv, sr)

        def wait_bufs(b):
            sv, dv, hv, tv, rv, _, sh, st, sr = bufs[b]
            pltpu.make_async_copy(xn.at[sv], hv, sh).wait()
            pltpu.make_async_copy(xn.at[dv], tv, st).wait()
            pltpu.make_async_copy(attr.at[pl.ds(0, _CHUNK)], rv, sr).wait()

        def compute(jc, b):
            _, _, hv, tv, rv, ov, _, _, _ = bufs[b]

            def edge_body(i4, carry2):
                for u in range(4):
                    e = i4 * 4 + u
                    acc0 = jnp.zeros((_L,), jnp.float32)
                    acc1 = jnp.zeros((_L,), jnp.float32)
                    for jj in range(_D // _L):
                        h = hv[e, pl.ds(jj * _L, _L)]
                        t = tv[e, pl.ds(jj * _L, _L)]
                        rh = rv[e, pl.ds(jj * _L, _L)]
                        rt = rv[e, pl.ds(_D + jj * _L, _L)]
                        term = jnp.abs(h * rh - t * rt)
                        if jj % 2 == 0:
                            acc0 = acc0 + term
                        else:
                            acc1 = acc1 + term
                    ov[e, pl.ds(0, _L)] = acc0 + acc1
                return carry2

            lax.fori_loop(0, _CHUNK // 4, edge_body, 0)
            base = wid * per_w + jc * _CHUNK
            pltpu.sync_copy(ov, out.at[pl.ds(base, _CHUNK)])

        issue(0, 0)

        def body(k, carry):
            jc = 2 * k
            wait_bufs(0)
            issue(jc + 1, 1)
            compute(jc, 0)
            wait_bufs(1)
            issue(jc + 2, 0)
            compute(jc + 1, 1)
            return carry

        lax.fori_loop(0, (n_chunks - 1) // 2, body, 0)
        wait_bufs(0)
        compute(n_chunks - 1, 0)

    return sc_partial


def kernel(x, edge_index, edge_attr):
    xn = _normalize(x.astype(jnp.float32))
    ei = edge_index.astype(jnp.int32).reshape(-1)
    partial = _make_sc_partial(edge_index.shape[1])(
        xn, ei, edge_attr.astype(jnp.float32))
    return _finish(partial)


# speedup vs baseline: 4.2331x; 1.3227x over previous
import functools

import jax
import jax.numpy as jnp
from jax import lax
from jax.experimental import pallas as pl
from jax.experimental.pallas import tpu as pltpu
from jax.experimental.pallas import tpu_sc as plsc

_GAMMA = 12.0
_D = 128
_CHUNK = 80

_info = plsc.get_sparse_core_info()
_NC = _info.num_cores
_NS = _info.num_subcores
_NW = _NC * _NS
_L = _info.num_lanes


def _normalize_body(x_ref, o_ref):
    x = x_ref[...]
    n = jnp.sqrt(jnp.sum(x * x, axis=1, keepdims=True))
    o_ref[...] = x / jnp.maximum(n, 1e-12)


def _normalize(x):
    n_rows, d = x.shape
    blk = 1000
    return pl.pallas_call(
        _normalize_body,
        grid=(n_rows // blk,),
        in_specs=[pl.BlockSpec((blk, d), lambda i: (i, 0))],
        out_specs=pl.BlockSpec((blk, d), lambda i: (i, 0)),
        out_shape=jax.ShapeDtypeStruct((n_rows, d), x.dtype),
    )(x)


_W = _D * _L


def _finish_body(p_ref, o_ref):
    x = p_ref[...]
    c = lax.broadcasted_iota(jnp.int32, (_W, _D), 0)
    g = lax.broadcasted_iota(jnp.int32, (_W, _D), 1)
    sel = jnp.where(c // _L == g, 1.0, 0.0).astype(jnp.float32)
    o_ref[...] = _GAMMA - jnp.dot(x, sel,
                                  precision=lax.Precision.HIGHEST,
                                  preferred_element_type=jnp.float32)


def _finish(partial_flat):
    n_rows = partial_flat.shape[0] // _W
    p2 = partial_flat.reshape(n_rows, _W)
    return pl.pallas_call(
        _finish_body,
        out_shape=jax.ShapeDtypeStruct((n_rows, _D), jnp.float32),
    )(p2)


def _make_sc_partial(num_edges):
    per_w = num_edges // _NW
    n_chunks = per_w // _CHUNK
    assert per_w * _NW == num_edges and n_chunks * _CHUNK == per_w
    assert n_chunks % 2 == 1

    mesh = plsc.VectorSubcoreMesh(core_axis_name="c", subcore_axis_name="s")

    buf_types = [
        pltpu.VMEM((_CHUNK,), jnp.int32),
        pltpu.VMEM((_CHUNK,), jnp.int32),
        pltpu.VMEM((_CHUNK, _D), jnp.float32),
        pltpu.VMEM((_CHUNK, _D), jnp.float32),
        pltpu.VMEM((_CHUNK, 2 * _D), jnp.float32),
        pltpu.VMEM((_CHUNK * _L,), jnp.float32),
    ]
    sem_types = [pltpu.SemaphoreType.DMA] * 6

    @functools.partial(
        pl.kernel,
        mesh=mesh,
        out_type=jax.ShapeDtypeStruct((num_edges * _L,), jnp.float32),
        scratch_types=buf_types + buf_types + sem_types,
    )
    def sc_partial(xn, ei, attr, out,
                   sv0, dv0, hv0, tv0, rv0, ov0,
                   sv1, dv1, hv1, tv1, rv1, ov1,
                   sh0, st0, sr0, sh1, st1, sr1):
        wid = lax.axis_index("s") * _NC + lax.axis_index("c")
        bufs = ((sv0, dv0, hv0, tv0, rv0, ov0, sh0, st0, sr0),
                (sv1, dv1, hv1, tv1, rv1, ov1, sh1, st1, sr1))

        def issue(jc, b):
            sv, dv, hv, tv, rv, _, sh, st, sr = bufs[b]
            base = wid * per_w + jc * _CHUNK
            pltpu.sync_copy(ei.at[pl.ds(base, _CHUNK)], sv)
            pltpu.sync_copy(ei.at[pl.ds(num_edges + base, _CHUNK)], dv)
            pltpu.async_copy(xn.at[sv], hv, sh)
            pltpu.async_copy(xn.at[dv], tv, st)
            pltpu.async_copy(attr.at[pl.ds(base, _CHUNK)], rv, sr)

        def wait_bufs(b):
            sv, dv, hv, tv, rv, _, sh, st, sr = bufs[b]
            pltpu.make_async_copy(xn.at[sv], hv, sh).wait()
            pltpu.make_async_copy(xn.at[dv], tv, st).wait()
            pltpu.make_async_copy(attr.at[pl.ds(0, _CHUNK)], rv, sr).wait()

        def compute(jc, b):
            _, _, hv, tv, rv, ov, _, _, _ = bufs[b]

            def edge_body(i4, carry2):
                for u in range(4):
                    e = i4 * 4 + u
                    acc0 = jnp.zeros((_L,), jnp.float32)
                    acc1 = jnp.zeros((_L,), jnp.float32)
                    for jj in range(_D // _L):
                        h = hv[e, pl.ds(jj * _L, _L)]
                        t = tv[e, pl.ds(jj * _L, _L)]
                        rh = rv[e, pl.ds(jj * _L, _L)]
                        rt = rv[e, pl.ds(_D + jj * _L, _L)]
                        term = jnp.abs(h * rh - t * rt)
                        if jj % 2 == 0:
                            acc0 = acc0 + term
                        else:
                            acc1 = acc1 + term
                    ov[pl.ds(e * _L, _L)] = acc0 + acc1
                return carry2

            lax.fori_loop(0, _CHUNK // 4, edge_body, 0)
            base = wid * per_w + jc * _CHUNK
            pltpu.sync_copy(ov, out.at[pl.ds(base * _L, _CHUNK * _L)])

        issue(0, 0)

        def body(k, carry):
            jc = 2 * k
            wait_bufs(0)
            issue(jc + 1, 1)
            compute(jc, 0)
            wait_bufs(1)
            issue(jc + 2, 0)
            compute(jc + 1, 1)
            return carry

        lax.fori_loop(0, (n_chunks - 1) // 2, body, 0)
        wait_bufs(0)
        compute(n_chunks - 1, 0)

    return sc_partial


def kernel(x, edge_index, edge_attr):
    xn = _normalize(x.astype(jnp.float32))
    ei = edge_index.astype(jnp.int32).reshape(-1)
    partial = _make_sc_partial(edge_index.shape[1])(
        xn, ei, edge_attr.astype(jnp.float32))
    return _finish(partial).reshape(-1, 1)


# speedup vs baseline: 5.5569x; 1.3127x over previous
import functools

import jax
import jax.numpy as jnp
from jax import lax
from jax.experimental import pallas as pl
from jax.experimental.pallas import tpu as pltpu
from jax.experimental.pallas import tpu_sc as plsc

_GAMMA = 12.0
_D = 128
_CHUNK = 80

_info = plsc.get_sparse_core_info()
_NC = _info.num_cores
_NS = _info.num_subcores
_NW = _NC * _NS
_L = _info.num_lanes


def _normalize_body(x_ref, o_ref):
    x = x_ref[...]
    n = jnp.sqrt(jnp.sum(x * x, axis=1, keepdims=True))
    o_ref[...] = x / jnp.maximum(n, 1e-12)


def _normalize(x):
    n_rows, d = x.shape
    blk = 1000
    return pl.pallas_call(
        _normalize_body,
        grid=(n_rows // blk,),
        in_specs=[pl.BlockSpec((blk, d), lambda i: (i, 0))],
        out_specs=pl.BlockSpec((blk, d), lambda i: (i, 0)),
        out_shape=jax.ShapeDtypeStruct((n_rows, d), x.dtype),
    )(x)


_W = _D * _L


def _finish_body(p_ref, o_ref):
    x = p_ref[...]
    c = lax.broadcasted_iota(jnp.int32, (_W, _D), 0)
    g = lax.broadcasted_iota(jnp.int32, (_W, _D), 1)
    sel = jnp.where(c // _L == g, 1.0, 0.0).astype(jnp.float32)
    o_ref[...] = _GAMMA - jnp.dot(x, sel,
                                  precision=lax.Precision.HIGHEST,
                                  preferred_element_type=jnp.float32)


def _finish(partial_flat):
    n_rows = partial_flat.shape[0] // _W
    p2 = partial_flat.reshape(n_rows, _W)
    return pl.pallas_call(
        _finish_body,
        out_shape=jax.ShapeDtypeStruct((n_rows, _D), jnp.float32),
    )(p2)


def _make_sc_partial(num_edges):
    per_w = num_edges // _NW
    n_chunks = per_w // _CHUNK
    assert per_w * _NW == num_edges and n_chunks * _CHUNK == per_w
    assert n_chunks % 2 == 1

    mesh = plsc.VectorSubcoreMesh(core_axis_name="c", subcore_axis_name="s")

    buf_types = [
        pltpu.VMEM((_CHUNK, _D), jnp.float32),
        pltpu.VMEM((_CHUNK, _D), jnp.float32),
        pltpu.VMEM((_CHUNK, 2 * _D), jnp.float32),
        pltpu.VMEM((_CHUNK * _L,), jnp.float32),
    ]
    sem_types = [pltpu.SemaphoreType.DMA] * 6

    @functools.partial(
        pl.kernel,
        mesh=mesh,
        out_type=jax.ShapeDtypeStruct((num_edges * _L,), jnp.float32),
        scratch_types=[pltpu.VMEM((per_w,), jnp.int32),
                       pltpu.VMEM((per_w,), jnp.int32)]
        + buf_types + buf_types + sem_types,
    )
    def sc_partial(xn, ei, attr, out, siv, div,
                   hv0, tv0, rv0, ov0,
                   hv1, tv1, rv1, ov1,
                   sh0, st0, sr0, sh1, st1, sr1):
        wid = lax.axis_index("s") * _NC + lax.axis_index("c")
        bufs = ((hv0, tv0, rv0, ov0, sh0, st0, sr0),
                (hv1, tv1, rv1, ov1, sh1, st1, sr1))

        pltpu.sync_copy(ei.at[pl.ds(wid * per_w, per_w)], siv)
        pltpu.sync_copy(ei.at[pl.ds(num_edges + wid * per_w, per_w)], div)

        def issue(jc, b):
            hv, tv, rv, _, sh, st, sr = bufs[b]
            base = wid * per_w + jc * _CHUNK
            off = jc * _CHUNK
            pltpu.async_copy(xn.at[siv.at[pl.ds(off, _CHUNK)]], hv, sh)
            pltpu.async_copy(xn.at[div.at[pl.ds(off, _CHUNK)]], tv, st)
            pltpu.async_copy(attr.at[pl.ds(base, _CHUNK)], rv, sr)

        def wait_bufs(b):
            hv, tv, rv, _, sh, st, sr = bufs[b]
            pltpu.make_async_copy(xn.at[siv.at[pl.ds(0, _CHUNK)]], hv, sh).wait()
            pltpu.make_async_copy(xn.at[div.at[pl.ds(0, _CHUNK)]], tv, st).wait()
            pltpu.make_async_copy(attr.at[pl.ds(0, _CHUNK)], rv, sr).wait()

        def compute(jc, b):
            hv, tv, rv, ov, _, _, _ = bufs[b]

            def edge_body(i4, carry2):
                for u in range(4):
                    e = i4 * 4 + u
                    acc0 = jnp.zeros((_L,), jnp.float32)
                    acc1 = jnp.zeros((_L,), jnp.float32)
                    for jj in range(_D // _L):
                        h = hv[e, pl.ds(jj * _L, _L)]
                        t = tv[e, pl.ds(jj * _L, _L)]
                        rh = rv[e, pl.ds(jj * _L, _L)]
                        rt = rv[e, pl.ds(_D + jj * _L, _L)]
                        term = jnp.abs(h * rh - t * rt)
                        if jj % 2 == 0:
                            acc0 = acc0 + term
                        else:
                            acc1 = acc1 + term
                    ov[pl.ds(e * _L, _L)] = acc0 + acc1
                return carry2

            lax.fori_loop(0, _CHUNK // 4, edge_body, 0)
            base = wid * per_w + jc * _CHUNK
            pltpu.sync_copy(ov, out.at[pl.ds(base * _L, _CHUNK * _L)])

        issue(0, 0)

        def body(k, carry):
            jc = 2 * k
            wait_bufs(0)
            issue(jc + 1, 1)
            compute(jc, 0)
            wait_bufs(1)
            issue(jc + 2, 0)
            compute(jc + 1, 1)
            return carry

        lax.fori_loop(0, (n_chunks - 1) // 2, body, 0)
        wait_bufs(0)
        compute(n_chunks - 1, 0)

    return sc_partial


def kernel(x, edge_index, edge_attr):
    xn = _normalize(x.astype(jnp.float32))
    ei = edge_index.astype(jnp.int32).reshape(-1)
    partial = _make_sc_partial(edge_index.shape[1])(
        xn, ei, edge_attr.astype(jnp.float32))
    return _finish(partial).reshape(-1, 1)
